# baseline (speedup 1.0000x reference)
import functools

import jax
import jax.numpy as jnp
from jax import lax
from jax.experimental import pallas as pl
from jax.experimental.pallas import tpu as pltpu
from jax.experimental.pallas import tpu_sc as plsc

_DIM = 256
_NEMB = 8192
_TILE = 256


def _nearest_body(xt_ref, e_ref, et_ref, idx_ref, diff_ref, acc_ref):
    i = pl.program_id(0)
    nsteps = pl.num_programs(0)
    xt = xt_ref[...]
    e = e_ref[...]
    et = et_ref[...]
    xbt = (2.0 * xt).astype(jnp.bfloat16)
    conv = lax.dot_general(
        e, xbt, (((0,), (0,)), ((), ())),
        preferred_element_type=jnp.float32,
    )
    xsq = jnp.sum(xt * xt, axis=0, keepdims=True)
    esq = jnp.sum(et * et, axis=1, keepdims=True)
    dist = (xsq - conv) + esq
    m = jnp.min(dist, axis=0, keepdims=True)
    row = lax.broadcasted_iota(jnp.int32, dist.shape, 0)
    idx = jnp.min(jnp.where(dist == m, row, _NEMB), axis=0)
    idx_ref[...] = idx.astype(jnp.int32)

    @pl.when(i == 0)
    def _init():
        acc_ref[0] = 0.0

    acc_ref[0] += jnp.sum(m)

    @pl.when(i == nsteps - 1)
    def _fin():
        diff_ref[0, 0] = acc_ref[0] / (nsteps * _TILE * _DIM)


def _nearest(flat_t, embed, embed_t):
    tok = flat_t.shape[1]
    grid = tok // _TILE
    return pl.pallas_call(
        _nearest_body,
        grid=(grid,),
        in_specs=[
            pl.BlockSpec((_DIM, _TILE), lambda i: (0, i)),
            pl.BlockSpec((_DIM, _NEMB), lambda i: (0, 0)),
            pl.BlockSpec((_NEMB, _DIM), lambda i: (0, 0)),
        ],
        out_specs=[
            pl.BlockSpec((_TILE,), lambda i: (i,)),
            pl.BlockSpec(memory_space=pltpu.SMEM),
        ],
        out_shape=[
            jax.ShapeDtypeStruct((tok,), jnp.int32),
            jax.ShapeDtypeStruct((1, 1), jnp.float32),
        ],
        scratch_shapes=[pltpu.SMEM((1,), jnp.float32)],
    )(flat_t, embed, embed_t)


def _gather_rows(table, idx):
    tok = idx.shape[0]
    dim = table.shape[1]
    info = plsc.get_sparse_core_info()
    nw = info.num_cores * info.num_subcores
    bpw = tok // nw
    ch = 128
    nch = bpw // ch
    idx2 = idx.reshape(nw * nch, ch)
    mesh = plsc.VectorSubcoreMesh(core_axis_name="c", subcore_axis_name="s")

    @functools.partial(
        pl.kernel,
        out_type=jax.ShapeDtypeStruct((tok, dim), jnp.float32),
        mesh=mesh,
        scratch_types=[
            pltpu.VMEM((nch, ch), jnp.int32),
            pltpu.VMEM((bpw, dim), jnp.float32),
            pltpu.SemaphoreType.DMA,
        ],
    )
    def k(table_hbm, idx_hbm, out_hbm, idx_v, rows_v, sem):
        wid = lax.axis_index("s") * info.num_cores + lax.axis_index("c")
        pltpu.sync_copy(idx_hbm.at[pl.ds(wid * nch, nch)], idx_v)
        copies = [
            pltpu.async_copy(
                table_hbm.at[idx_v.at[j]], rows_v.at[pl.ds(j * ch, ch)], sem
            )
            for j in range(nch)
        ]
        for c in copies:
            c.wait()
        pltpu.sync_copy(rows_v, out_hbm.at[pl.ds(wid * bpw, bpw)])

    return k(table, idx2)


def kernel(input_data, embed):
    flat_t = input_data.reshape(-1, _DIM).T
    embed_t = embed.T
    idx_flat, diff11 = _nearest(flat_t, embed, embed_t)
    quant_flat = _gather_rows(embed_t, idx_flat)
    quantize = quant_flat.reshape(input_data.shape)
    embed_ind = idx_flat.reshape(input_data.shape[:-1])
    return (quantize, diff11[0, 0], embed_ind)
